# Initial kernel scaffold; baseline (speedup 1.0000x reference)
import jax
import jax.numpy as jnp
from jax.experimental import pallas as pl


def kernel(x, edge_index, W1, b1, W2, b2):
    raise NotImplementedError("write your pallas kernel here")


# speedup vs baseline: 12.6938x; 12.6938x over previous
import functools

import jax
import jax.numpy as jnp
from jax import lax
from jax.experimental import pallas as pl
from jax.experimental.pallas import tpu as pltpu
from jax.experimental.pallas import tpu_sc as plsc

_NC = 2
_NS = 16
_CHUNK = 128
_LANES = 16


def _sc_mesh():
    return plsc.VectorSubcoreMesh(core_axis_name="c", subcore_axis_name="s")


def _sc_degree(dst, zeros_flat, np_):
    e = dst.shape[0]
    nw = _NC * _NS
    nchunk = e // _CHUNK
    niter = -(-nchunk // nw)
    npw = np_ // _NS

    @functools.partial(
        pl.kernel,
        out_type=jax.ShapeDtypeStruct((_NC, np_), jnp.float32),
        mesh=_sc_mesh(),
        compiler_params=pltpu.CompilerParams(needs_layout_passes=False),
        scratch_types=[
            pltpu.VMEM((_CHUNK,), jnp.int32),
            pltpu.VMEM((np_,), jnp.float32),
            pltpu.VMEM((_NS, npw), jnp.float32),
            pltpu.VMEM_SHARED((_NS, np_), jnp.float32),
        ],
    )
    def k(dst_hbm, z_hbm, out_hbm, idx_v, hist_v, red_v, stage_sh):
        c = lax.axis_index("c")
        s = lax.axis_index("s")
        w = c * _NS + s
        pltpu.sync_copy(z_hbm, hist_v)
        ones = jnp.full((_LANES,), 1.0, jnp.float32)

        def chunk_body(i, carry):
            ch = w + i * nw

            @pl.when(ch < nchunk)
            def _():
                pltpu.sync_copy(dst_hbm.at[pl.ds(ch * _CHUNK, _CHUNK)], idx_v)

                def inner(j, carry2):
                    dvec = idx_v[pl.ds(j * _LANES, _LANES)]
                    plsc.addupdate_scatter(hist_v, (dvec,), ones)
                    return carry2

                lax.fori_loop(0, _CHUNK // _LANES, inner, 0)

            return carry

        lax.fori_loop(0, niter, chunk_body, 0)

        pltpu.sync_copy(hist_v, stage_sh.at[s])
        plsc.subcore_barrier()
        for r in range(_NS):
            pltpu.sync_copy(stage_sh.at[r, pl.ds(s * npw, npw)], red_v.at[r])

        def red_body(j, carry):
            acc = red_v[0, pl.ds(j * _LANES, _LANES)]
            for r in range(1, _NS):
                acc = acc + red_v[r, pl.ds(j * _LANES, _LANES)]
            hist_v[pl.ds(j * _LANES, _LANES)] = acc
            return carry

        lax.fori_loop(0, npw // _LANES, red_body, 0)
        pltpu.sync_copy(hist_v.at[pl.ds(0, npw)],
                        out_hbm.at[c, pl.ds(s * npw, npw)])

    return k(dst, zeros_flat)


def _sc_scatter(h, src, dst, zeros_slab, np_):
    e = src.shape[0]
    dh = h.shape[1]
    nw = _NC * _NS
    nchunk = e // _CHUNK
    niter = -(-nchunk // nw)
    rows_per = np_ // _NS

    @functools.partial(
        pl.kernel,
        out_type=jax.ShapeDtypeStruct((_NC, np_, dh), jnp.float32),
        mesh=_sc_mesh(),
        compiler_params=pltpu.CompilerParams(needs_layout_passes=False),
        scratch_types=[
            pltpu.VMEM((_CHUNK,), jnp.int32),
            pltpu.VMEM((_CHUNK,), jnp.int32),
            pltpu.VMEM((_CHUNK, dh), jnp.float32),
            pltpu.VMEM_SHARED((np_, dh), jnp.float32),
            pltpu.SemaphoreType.DMA,
        ],
    )
    def k(h_hbm, src_hbm, dst_hbm, z_hbm, out_hbm, isrc, idst, rows, acc_sh, sem):
        c = lax.axis_index("c")
        s = lax.axis_index("s")
        w = c * _NS + s
        pltpu.sync_copy(z_hbm, acc_sh.at[pl.ds(s * rows_per, rows_per)])
        plsc.subcore_barrier()

        def body(i, carry):
            ch = w + i * nw

            @pl.when(ch < nchunk)
            def _():
                pltpu.sync_copy(src_hbm.at[pl.ds(ch * _CHUNK, _CHUNK)], isrc)
                pltpu.sync_copy(dst_hbm.at[pl.ds(ch * _CHUNK, _CHUNK)], idst)
                pltpu.async_copy(h_hbm.at[isrc], rows, sem).wait()
                pltpu.sync_copy(rows, acc_sh.at[idst], add=True)

            return carry

        lax.fori_loop(0, niter, body, 0)
        plsc.subcore_barrier()
        pltpu.sync_copy(acc_sh.at[pl.ds(s * rows_per, rows_per)],
                        out_hbm.at[c, pl.ds(s * rows_per, rows_per)])

    return k(h, src, dst, zeros_slab)


def _tc_layer1(x_pad, W1, deg3, r_blk):
    np_, d_in = x_pad.shape
    d_h = W1.shape[1]
    half = d_h // 2
    grid = np_ // r_blk

    def body(x_ref, w_ref, deg_ref, lo_ref, hi_ref, dis_ref):
        deg = deg_ref[...]
        dis = lax.rsqrt(deg[0] + deg[1] + 1.0)
        h = jnp.dot(x_ref[...], w_ref[...], preferred_element_type=jnp.float32)
        lo_ref[...] = h[:, :half] * dis
        hi_ref[...] = h[:, half:] * dis
        dis_ref[...] = dis

    return pl.pallas_call(
        body,
        grid=(grid,),
        in_specs=[
            pl.BlockSpec((r_blk, d_in), lambda i: (i, 0)),
            pl.BlockSpec((d_in, d_h), lambda i: (0, 0)),
            pl.BlockSpec((_NC, r_blk, 1), lambda i: (0, i, 0)),
        ],
        out_specs=[
            pl.BlockSpec((r_blk, half), lambda i: (i, 0)),
            pl.BlockSpec((r_blk, half), lambda i: (i, 0)),
            pl.BlockSpec((r_blk, 1), lambda i: (i, 0)),
        ],
        out_shape=[
            jax.ShapeDtypeStruct((np_, half), jnp.float32),
            jax.ShapeDtypeStruct((np_, half), jnp.float32),
            jax.ShapeDtypeStruct((np_, 1), jnp.float32),
        ],
    )(x_pad, W1, deg3)


def _tc_layer2(acc_lo, acc_hi, h_lo, h_hi, dis, b1, W2, r_blk):
    np_ = h_lo.shape[0]
    half = h_lo.shape[1]
    d_out = W2.shape[1]
    grid = np_ // r_blk

    def body(alo_ref, ahi_ref, hlo_ref, hhi_ref, dis_ref, b1_ref, w2_ref,
             out_ref):
        dis_v = dis_ref[...]
        alo = alo_ref[...]
        ahi = ahi_ref[...]
        b1v = b1_ref[...]
        zlo = jnp.maximum((alo[0] + alo[1] + hlo_ref[...]) * dis_v
                          + b1v[:, :half], 0.0)
        zhi = jnp.maximum((ahi[0] + ahi[1] + hhi_ref[...]) * dis_v
                          + b1v[:, half:], 0.0)
        w2 = w2_ref[...]
        h2 = (jnp.dot(zlo, w2[:half], preferred_element_type=jnp.float32)
              + jnp.dot(zhi, w2[half:], preferred_element_type=jnp.float32))
        out_ref[...] = h2 * dis_v

    acc_spec = pl.BlockSpec((_NC, r_blk, half), lambda i: (0, i, 0))
    row_spec = pl.BlockSpec((r_blk, half), lambda i: (i, 0))
    return pl.pallas_call(
        body,
        grid=(grid,),
        in_specs=[
            acc_spec,
            acc_spec,
            row_spec,
            row_spec,
            pl.BlockSpec((r_blk, 1), lambda i: (i, 0)),
            pl.BlockSpec((1, 2 * half), lambda i: (0, 0)),
            pl.BlockSpec((2 * half, d_out), lambda i: (0, 0)),
        ],
        out_specs=pl.BlockSpec((r_blk, d_out), lambda i: (i, 0)),
        out_shape=jax.ShapeDtypeStruct((np_, d_out), jnp.float32),
    )(acc_lo, acc_hi, h_lo, h_hi, dis, b1, W2)


def _tc_final(acc2, h2, dis, b2, r_blk):
    np_, d_out = h2.shape
    grid = np_ // r_blk

    def body(a_ref, h_ref, dis_ref, b_ref, out_ref):
        a = a_ref[...]
        out_ref[...] = (a[0] + a[1] + h_ref[...]) * dis_ref[...] + b_ref[...]

    return pl.pallas_call(
        body,
        grid=(grid,),
        in_specs=[
            pl.BlockSpec((_NC, r_blk, d_out), lambda i: (0, i, 0)),
            pl.BlockSpec((r_blk, d_out), lambda i: (i, 0)),
            pl.BlockSpec((r_blk, 1), lambda i: (i, 0)),
            pl.BlockSpec((1, d_out), lambda i: (0, 0)),
        ],
        out_specs=pl.BlockSpec((r_blk, d_out), lambda i: (i, 0)),
        out_shape=jax.ShapeDtypeStruct((np_, d_out), jnp.float32),
    )(acc2, h2, dis, b2)


def kernel(x, edge_index, W1, b1, W2, b2):
    n, d_in = x.shape
    d_h = W1.shape[1]
    e = edge_index.shape[1]
    assert e % _CHUNK == 0 and d_h % 256 == 0 and d_in % 128 == 0
    np_ = -(-n // 2048) * 2048
    r_blk = 1024

    src = edge_index[0]
    dst = edge_index[1]
    zeros_flat = jnp.zeros((np_,), jnp.float32)
    zeros_slab = jnp.zeros((np_ // _NS, d_h // 2), jnp.float32)

    deg = _sc_degree(dst, zeros_flat, np_)
    deg3 = deg.reshape(_NC, np_, 1)
    x_pad = jnp.pad(x, ((0, np_ - n), (0, 0)))
    h_lo, h_hi, dis = _tc_layer1(x_pad, W1, deg3, r_blk)
    acc_lo = _sc_scatter(h_lo, src, dst, zeros_slab, np_)
    acc_hi = _sc_scatter(h_hi, src, dst, zeros_slab, np_)
    h2 = _tc_layer2(acc_lo, acc_hi, h_lo, h_hi, dis,
                    b1.reshape(1, -1), W2, r_blk)
    acc2 = _sc_scatter(h2, src, dst, zeros_slab, np_)
    out = _tc_final(acc2, h2, dis, b2.reshape(1, -1), r_blk)
    return out[:n]
